# baseline (speedup 1.0000x reference)
import functools

import jax
import jax.numpy as jnp
from jax import lax
from jax.experimental import pallas as pl
from jax.experimental.pallas import tpu as pltpu
from jax.experimental.pallas import tpu_sc as plsc

_D = 64
_NC, _NS = 2, 16
_NW = _NC * _NS
_CH = 800


@functools.lru_cache(maxsize=None)
def _make_gather(b, l, d):
    n_total = b * l
    bpw = n_total // _NW
    ch = _CH
    nch = bpw // ch
    rows_per_chunk = ch // l
    assert nch * ch == bpw and rows_per_chunk * l == ch

    mesh = plsc.VectorSubcoreMesh(core_axis_name="c", subcore_axis_name="s")

    @functools.partial(
        pl.kernel,
        mesh=mesh,
        compiler_params=pltpu.CompilerParams(use_tc_tiling_on_sc=False),
        out_type=jax.ShapeDtypeStruct((b, l, d), jnp.float32),
        scratch_types=[
            pltpu.VMEM((bpw,), jnp.int32),
            pltpu.VMEM((2, ch, d), jnp.float32),
            pltpu.SemaphoreType.DMA,
            pltpu.SemaphoreType.DMA,
        ],
    )
    def emb(table_hbm, idx_hbm, out_hbm, idx_v, rows_v, sem0, sem1):
        wid = lax.axis_index("s") * _NC + lax.axis_index("c")
        base = wid * bpw
        pltpu.sync_copy(idx_hbm.at[pl.ds(base, bpw)], idx_v)
        sems = (sem0, sem1)
        copies = [None, None]
        copies[0] = pltpu.async_copy(
            table_hbm.at[idx_v.at[pl.ds(0, ch)]], rows_v.at[0], sems[0])
        for i in range(nch):
            cur = i % 2
            nxt = 1 - cur
            if i + 1 < nch:
                copies[nxt] = pltpu.async_copy(
                    table_hbm.at[idx_v.at[pl.ds((i + 1) * ch, ch)]],
                    rows_v.at[nxt], sems[nxt])
            copies[cur].wait()
            brow0 = (base + i * ch) // l
            for k in range(rows_per_chunk):
                pltpu.sync_copy(rows_v.at[cur, pl.ds(k * l, l)],
                                out_hbm.at[brow0 + k])

    return emb


def kernel(token_ids, weight):
    b, l = token_ids.shape
    idx = token_ids.reshape(b * l).astype(jnp.int32)
    z = (token_ids[0, 0] - token_ids[0, 0]).astype(jnp.float32)
    return _make_gather(b, l, weight.shape[1])(weight + z, idx)
